# Initial kernel scaffold; baseline (speedup 1.0000x reference)
import jax
import jax.numpy as jnp
from jax.experimental import pallas as pl


def kernel(u, y, ind, U, Y):
    raise NotImplementedError("write your pallas kernel here")


# speedup vs baseline: 1.1728x; 1.1728x over previous
import jax
import jax.numpy as jnp
from jax.experimental import pallas as pl

_NT = 50000
_B = 1024
_BIT = 32
_NC = 10
_ETA = 0.001
_CB = 1000
_NJ = _NT // _CB


def _f(ip, s):
    a = jnp.abs(ip)
    return jnp.log1p(jnp.exp(-a)) + jnp.maximum(ip, 0.0) - s * ip


def _loss_kernel(u_ref, y_ref, indc_ref, indr_ref, U_ref, Y_ref, out_ref):
    j = pl.program_id(0)
    u16 = u_ref[...].astype(jnp.bfloat16)
    y16 = y_ref[...].astype(jnp.bfloat16)
    Ub = U_ref[...].astype(jnp.bfloat16)
    Yb = Y_ref[...].astype(jnp.bfloat16)
    ind_c = indc_ref[...]

    dn = (((1,), (1,)), ((), ()))
    ip = jax.lax.dot_general(u16, Ub, dn, preferred_element_type=jnp.float32) * 0.5
    sd = jax.lax.dot_general(y16, Yb, dn, preferred_element_type=jnp.float32)
    s = (sd > 0).astype(jnp.float32)
    colsum = jnp.sum(_f(ip, s), axis=0, keepdims=True)

    iota_c = jax.lax.broadcasted_iota(jnp.int32, (_B, _CB), 1) + j * _CB
    touched = jnp.max((ind_c == iota_c).astype(jnp.float32), axis=0, keepdims=True)
    contrib = jnp.sum(colsum * (1.0 - touched))

    @pl.when(j == 0)
    def _first():
        ind_r = indr_ref[...]
        ii = jax.lax.broadcasted_iota(jnp.int32, (_B, _B), 0)
        jj = jax.lax.broadcasted_iota(jnp.int32, (_B, _B), 1)
        superseded = jnp.max(
            jnp.where((ind_c == ind_r) & (ii > jj), 1.0, 0.0), axis=0, keepdims=True
        )
        a_new = jax.lax.dot_general(u16, u16, dn, preferred_element_type=jnp.float32) * 0.5
        s_new = (jax.lax.dot_general(y16, y16, dn, preferred_element_type=jnp.float32) > 0
                 ).astype(jnp.float32)
        new_cols = jnp.sum(_f(a_new, s_new), axis=0, keepdims=True)
        c_new = jnp.sum(new_cols * (1.0 - superseded))
        u = u_ref[...]
        quant = jnp.sum((u - jnp.sign(u)) ** 2) * (_ETA * _NT / _BIT)
        out_ref[...] = jnp.full((1, 1), contrib + c_new + quant, jnp.float32)

    @pl.when(j != 0)
    def _rest():
        out_ref[...] = out_ref[...] + contrib


def kernel(u, y, ind, U, Y):
    ind = ind.astype(jnp.int32)
    ind_c = ind.reshape(_B, 1)
    ind_r = ind.reshape(1, _B)
    acc = pl.pallas_call(
        _loss_kernel,
        grid=(_NJ,),
        in_specs=[
            pl.BlockSpec((_B, _BIT), lambda j: (0, 0)),
            pl.BlockSpec((_B, _NC), lambda j: (0, 0)),
            pl.BlockSpec((_B, 1), lambda j: (0, 0)),
            pl.BlockSpec((1, _B), lambda j: (0, 0)),
            pl.BlockSpec((_CB, _BIT), lambda j: (j, 0)),
            pl.BlockSpec((_CB, _NC), lambda j: (j, 0)),
        ],
        out_specs=pl.BlockSpec((1, 1), lambda j: (0, 0)),
        out_shape=jax.ShapeDtypeStruct((1, 1), jnp.float32),
    )(u, y, ind_c, ind_r, U, Y)
    return acc[0, 0] / (_B * _NT)


# speedup vs baseline: 1.5048x; 1.2831x over previous
import jax
import jax.numpy as jnp
from jax.experimental import pallas as pl

_NT = 50000
_B = 1024
_BIT = 32
_NC = 10
_ETA = 0.001
_CB = 1000
_NJ = _NT // _CB

_LOG2E = 1.4426950408889634
_LN2 = 0.6931471805599453
_DN = (((1,), (1,)), ((), ()))


def _colsums(ip, sd):
    t = jnp.minimum(ip, -ip) * _LOG2E
    lg = jnp.log2(1.0 + jnp.exp2(t))
    cs_g = jnp.sum(lg - 0.5 * t, axis=0, keepdims=True) * _LN2
    cs_sip = jnp.sum(jnp.where(sd > 0, ip, 0.0), axis=0, keepdims=True)
    return cs_g, cs_sip


def _loss_kernel(uh_ref, y_ref, ush_ref, indc_ref, indr_ref, U_ref, Y_ref, out_ref):
    j = pl.program_id(0)
    uh16 = uh_ref[...].astype(jnp.bfloat16)
    y16 = y_ref[...].astype(jnp.bfloat16)
    Ub = U_ref[...]
    Yb = Y_ref[...]
    ind_c = indc_ref[...]

    ip = jax.lax.dot_general(uh16, Ub.astype(jnp.bfloat16), _DN,
                             preferred_element_type=jnp.float32)
    sd = jax.lax.dot_general(y16, Yb.astype(jnp.bfloat16), _DN,
                             preferred_element_type=jnp.float32)
    cs_g, cs_sip = _colsums(ip, sd)
    cs_ip = jax.lax.dot_general(ush_ref[...], Ub, _DN,
                                preferred_element_type=jnp.float32)
    cs_f = cs_g + 0.5 * cs_ip - cs_sip

    iota_c = jax.lax.broadcasted_iota(jnp.int32, (_B, _CB), 1) + j * _CB
    untouched = jnp.min(
        jnp.where(ind_c == iota_c, 0.0, 1.0), axis=0, keepdims=True)
    contrib = jnp.sum(cs_f * untouched)

    @pl.when(j == 0)
    def _first():
        ind_r = indr_ref[...]
        ii = jax.lax.broadcasted_iota(jnp.int32, (_B, _B), 0)
        jj = jax.lax.broadcasted_iota(jnp.int32, (_B, _B), 1)
        winner = jnp.min(
            jnp.where((ind_c == ind_r) & (ii > jj), 0.0, 1.0),
            axis=0, keepdims=True)
        u = uh_ref[...] * 2.0
        ipn = jax.lax.dot_general(uh16, u.astype(jnp.bfloat16), _DN,
                                  preferred_element_type=jnp.float32)
        sdn = jax.lax.dot_general(y16, y16, _DN,
                                  preferred_element_type=jnp.float32)
        cs_gn, cs_sipn = _colsums(ipn, sdn)
        cs_ipn = jax.lax.dot_general(ush_ref[...], u, _DN,
                                     preferred_element_type=jnp.float32)
        c_new = jnp.sum((cs_gn + 0.5 * cs_ipn - cs_sipn) * winner)
        quant = jnp.sum((u - jnp.sign(u)) ** 2) * (_ETA * _NT / _BIT)
        out_ref[...] = jnp.full((1, 1), contrib + c_new + quant, jnp.float32)

    @pl.when(j != 0)
    def _rest():
        out_ref[...] = out_ref[...] + contrib


def kernel(u, y, ind, U, Y):
    ind = ind.astype(jnp.int32)
    ind_c = ind.reshape(_B, 1)
    ind_r = ind.reshape(1, _B)
    uh = u * 0.5
    ush = jnp.sum(uh, axis=0, keepdims=True)
    acc = pl.pallas_call(
        _loss_kernel,
        grid=(_NJ,),
        in_specs=[
            pl.BlockSpec((_B, _BIT), lambda j: (0, 0)),
            pl.BlockSpec((_B, _NC), lambda j: (0, 0)),
            pl.BlockSpec((1, _BIT), lambda j: (0, 0)),
            pl.BlockSpec((_B, 1), lambda j: (0, 0)),
            pl.BlockSpec((1, _B), lambda j: (0, 0)),
            pl.BlockSpec((_CB, _BIT), lambda j: (j, 0)),
            pl.BlockSpec((_CB, _NC), lambda j: (j, 0)),
        ],
        out_specs=pl.BlockSpec((1, 1), lambda j: (0, 0)),
        out_shape=jax.ShapeDtypeStruct((1, 1), jnp.float32),
    )(uh, y, ush, ind_c, ind_r, U, Y)
    return acc[0, 0] / (_B * _NT)


# speedup vs baseline: 1.6904x; 1.1234x over previous
import jax
import jax.numpy as jnp
from jax.experimental import pallas as pl

_NT = 50000
_B = 1024
_BIT = 32
_NC = 10
_ETA = 0.001
_CB = 2000
_NJ = _NT // _CB

_LOG2E = 1.4426950408889634
_LN2 = 0.6931471805599453
_DN = (((1,), (1,)), ((), ()))


def _colsums(ip, sd):
    a = jnp.abs(ip)
    lg = jnp.log2(1.0 + jnp.exp2(a * (-_LOG2E)))
    cs_g = (jnp.sum(lg, axis=0, keepdims=True) * _LN2
            + 0.5 * jnp.sum(a, axis=0, keepdims=True))
    cs_sip = jnp.sum(jnp.where(sd > 0, ip, 0.0), axis=0, keepdims=True)
    return cs_g, cs_sip


def _loss_kernel(uh_ref, y_ref, ush_ref, indc_ref, indr_ref, U_ref, Y_ref, out_ref):
    j = pl.program_id(0)
    uh16 = uh_ref[...].astype(jnp.bfloat16)
    y16 = y_ref[...].astype(jnp.bfloat16)
    Ub = U_ref[...]
    Yb = Y_ref[...]
    ind_c = indc_ref[...]

    ip = jax.lax.dot_general(uh16, Ub.astype(jnp.bfloat16), _DN,
                             preferred_element_type=jnp.float32)
    sd = jax.lax.dot_general(y16, Yb.astype(jnp.bfloat16), _DN,
                             preferred_element_type=jnp.float32)
    cs_g, cs_sip = _colsums(ip, sd)
    cs_ip = jax.lax.dot_general(ush_ref[...], Ub, _DN,
                                preferred_element_type=jnp.float32)
    cs_f = cs_g + 0.5 * cs_ip - cs_sip

    iota_c = jax.lax.broadcasted_iota(jnp.int32, (_B, _CB), 1) + j * _CB
    untouched = jnp.min(
        jnp.where(ind_c == iota_c, 0.0, 1.0), axis=0, keepdims=True)
    contrib = jnp.sum(cs_f * untouched)

    @pl.when(j == 0)
    def _first():
        ind_r = indr_ref[...]
        ii = jax.lax.broadcasted_iota(jnp.int32, (_B, _B), 0)
        jj = jax.lax.broadcasted_iota(jnp.int32, (_B, _B), 1)
        winner = jnp.min(
            jnp.where((ind_c == ind_r) & (ii > jj), 0.0, 1.0),
            axis=0, keepdims=True)
        u = uh_ref[...] * 2.0
        ipn = jax.lax.dot_general(uh16, u.astype(jnp.bfloat16), _DN,
                                  preferred_element_type=jnp.float32)
        sdn = jax.lax.dot_general(y16, y16, _DN,
                                  preferred_element_type=jnp.float32)
        cs_gn, cs_sipn = _colsums(ipn, sdn)
        cs_ipn = jax.lax.dot_general(ush_ref[...], u, _DN,
                                     preferred_element_type=jnp.float32)
        c_new = jnp.sum((cs_gn + 0.5 * cs_ipn - cs_sipn) * winner)
        quant = jnp.sum((u - jnp.sign(u)) ** 2) * (_ETA * _NT / _BIT)
        out_ref[...] = jnp.full((1, 1), contrib + c_new + quant, jnp.float32)

    @pl.when(j != 0)
    def _rest():
        out_ref[...] = out_ref[...] + contrib


def kernel(u, y, ind, U, Y):
    ind = ind.astype(jnp.int32)
    ind_c = ind.reshape(_B, 1)
    ind_r = ind.reshape(1, _B)
    uh = u * 0.5
    ush = jnp.sum(uh, axis=0, keepdims=True)
    acc = pl.pallas_call(
        _loss_kernel,
        grid=(_NJ,),
        in_specs=[
            pl.BlockSpec((_B, _BIT), lambda j: (0, 0)),
            pl.BlockSpec((_B, _NC), lambda j: (0, 0)),
            pl.BlockSpec((1, _BIT), lambda j: (0, 0)),
            pl.BlockSpec((_B, 1), lambda j: (0, 0)),
            pl.BlockSpec((1, _B), lambda j: (0, 0)),
            pl.BlockSpec((_CB, _BIT), lambda j: (j, 0)),
            pl.BlockSpec((_CB, _NC), lambda j: (j, 0)),
        ],
        out_specs=pl.BlockSpec((1, 1), lambda j: (0, 0)),
        out_shape=jax.ShapeDtypeStruct((1, 1), jnp.float32),
    )(uh, y, ush, ind_c, ind_r, U, Y)
    return acc[0, 0] / (_B * _NT)
